# baseline (speedup 1.0000x reference)
import jax
import jax.numpy as jnp
from jax.experimental import pallas as pl
from jax.experimental.pallas import tpu as pltpu

_HIDDEN = 768
_EXPERTS = 64
_BLOCK_M = 4096


def _router_kernel(x_ref, w_ref, o_ref, wb_ref):
    @pl.when(pl.program_id(0) == 0)
    def _():
        wb_ref[...] = w_ref[...].astype(jnp.bfloat16)

    o_ref[...] = jax.lax.dot_general(
        wb_ref[...],
        x_ref[...].astype(jnp.bfloat16),
        dimension_numbers=(((1,), (1,)), ((), ())),
        preferred_element_type=jnp.float32,
    )


@jax.jit
def kernel(hidden_states, W):
    x = hidden_states.reshape(-1, _HIDDEN)
    m = x.shape[0]
    grid = (m // _BLOCK_M,)
    out_t = pl.pallas_call(
        _router_kernel,
        grid=grid,
        in_specs=[
            pl.BlockSpec((_BLOCK_M, _HIDDEN), lambda i: (i, 0)),
            pl.BlockSpec((_EXPERTS, _HIDDEN), lambda i: (0, 0)),
        ],
        out_specs=pl.BlockSpec((_EXPERTS, _BLOCK_M), lambda i: (0, i)),
        out_shape=jax.ShapeDtypeStruct((_EXPERTS, m), jnp.float32),
        scratch_shapes=[pltpu.VMEM((_EXPERTS, _HIDDEN), jnp.bfloat16)],
        compiler_params=pltpu.CompilerParams(
            dimension_semantics=("parallel",),
        ),
    )(x, W)
    return out_t.T
